# Initial kernel scaffold; baseline (speedup 1.0000x reference)
import jax
import jax.numpy as jnp
from jax.experimental import pallas as pl


def kernel(x, batch, W_att, b_att, context, W_out, b_out):
    raise NotImplementedError("write your pallas kernel here")


# speedup vs baseline: 15.3293x; 15.3293x over previous
import functools

import jax
import jax.numpy as jnp
from jax.experimental import pallas as pl
from jax.experimental.pallas import tpu as pltpu

_NUM_SEGMENTS = 128


def _body(x_ref, b_ref, wa_ref, ba_ref, cx_ref, wo_ref, bo_ref, out_ref,
          m_ref, z_ref, acc_ref):
    i = pl.program_id(0)
    num_blocks = pl.num_programs(0)
    neg_inf = jnp.float32(-jnp.inf)
    num_seg = m_ref.shape[1]

    @pl.when(i == 0)
    def _init():
        m_ref[...] = jnp.full(m_ref.shape, neg_inf, jnp.float32)
        z_ref[...] = jnp.zeros(z_ref.shape, jnp.float32)
        acc_ref[...] = jnp.zeros(acc_ref.shape, jnp.float32)

    xb = x_ref[...]
    logits = jax.lax.dot_general(
        xb, wa_ref[...], (((1,), (1,)), ((), ()))) + ba_ref[...]
    t = jnp.tanh(logits)
    s = jnp.sum(t * cx_ref[...], axis=1, keepdims=True)

    bv = b_ref[0]
    seg = jax.lax.broadcasted_iota(jnp.int32, (1, num_seg), 1)
    onehot = bv == seg

    m_blk = jnp.max(jnp.where(onehot, s, neg_inf), axis=0, keepdims=True)
    m_old = m_ref[...]
    m_new = jnp.maximum(m_old, m_blk)
    scale = jnp.where(m_new == neg_inf, 1.0, jnp.exp(m_old - m_new))

    m_row = jnp.max(jnp.where(onehot, m_new, neg_inf), axis=1, keepdims=True)
    e = jnp.exp(s - m_row)
    w = jnp.where(onehot, e, 0.0)

    z_ref[...] = z_ref[...] * scale + jnp.sum(w, axis=0, keepdims=True)
    scale_c = jnp.transpose(scale)
    acc_ref[...] = acc_ref[...] * scale_c + jax.lax.dot_general(
        w, xb, (((0,), (0,)), ((), ())))
    m_ref[...] = m_new

    @pl.when(i == num_blocks - 1)
    def _finish():
        zc = jnp.transpose(z_ref[...])
        rep = acc_ref[...] / (zc + 1e-8)
        out_ref[...] = jax.lax.dot_general(
            rep, wo_ref[...], (((1,), (1,)), ((), ()))) + bo_ref[...]


def _pick_block(n):
    for k in range(min(n, 2048), 7, -1):
        if n % k == 0 and k % 8 == 0:
            return k
    return None


@functools.partial(jax.jit, static_argnames=("num_segments", "interpret"))
def _pooled_attention(x, batch, W_att, b_att, context, W_out, b_out,
                      num_segments=_NUM_SEGMENTS, interpret=False):
    n, d = x.shape
    a = W_att.shape[0]
    k = _pick_block(n)
    if k is None:
        k = min(2048, 8 * ((n + 7) // 8))
        n_pad = ((n + k - 1) // k) * k
        x = jnp.pad(x, ((0, n_pad - n), (0, 0)))
        batch = jnp.pad(batch, (0, n_pad - n), constant_values=-1)
        n = n_pad
    g = n // k

    batch3 = batch.reshape(g, k, 1)
    ba2 = b_att.reshape(1, a)
    cx2 = context.reshape(1, a)
    bo2 = b_out.reshape(1, d)

    out = pl.pallas_call(
        _body,
        grid=(g,),
        in_specs=[
            pl.BlockSpec((k, d), lambda i: (i, 0)),
            pl.BlockSpec((1, k, 1), lambda i: (i, 0, 0)),
            pl.BlockSpec((a, d), lambda i: (0, 0)),
            pl.BlockSpec((1, a), lambda i: (0, 0)),
            pl.BlockSpec((1, a), lambda i: (0, 0)),
            pl.BlockSpec((d, d), lambda i: (0, 0)),
            pl.BlockSpec((1, d), lambda i: (0, 0)),
        ],
        out_specs=pl.BlockSpec((num_segments, d), lambda i: (0, 0)),
        out_shape=jax.ShapeDtypeStruct((num_segments, d), jnp.float32),
        scratch_shapes=[
            pltpu.VMEM((1, num_segments), jnp.float32),
            pltpu.VMEM((1, num_segments), jnp.float32),
            pltpu.VMEM((num_segments, d), jnp.float32),
        ],
        compiler_params=pltpu.CompilerParams(
            dimension_semantics=("arbitrary",)),
        interpret=interpret,
    )(x, batch3, W_att, ba2, cx2, W_out, bo2)
    return out


def kernel(x, batch, W_att, b_att, context, W_out, b_out):
    return _pooled_attention(x, batch, W_att, b_att, context, W_out, b_out)
